# Initial kernel scaffold; baseline (speedup 1.0000x reference)
import jax
import jax.numpy as jnp
from jax.experimental import pallas as pl


def kernel(z, weight):
    raise NotImplementedError("write your pallas kernel here")


# speedup vs baseline: 1.4568x; 1.4568x over previous
import functools

import jax
import jax.numpy as jnp
from jax.experimental import pallas as pl
from jax.experimental.pallas import tpu as pltpu

NUM_CODES = 512
LATENT_DIM = 384
BETA = 0.25
N_TOKENS = 32768
BN = 1024
G = N_TOKENS // BN


def _body(z_ref, wn_ref, s_ref, zq_ref, idx_ref, loss_ref):
    i = pl.program_id(0)

    @pl.when(i == 0)
    def _init():
        loss_ref[...] = jnp.zeros((1, 1), jnp.float32)

    wn = wn_ref[...]
    z = z_ref[...]
    zn = z / jnp.maximum(jnp.sqrt(jnp.sum(z * z, axis=1, keepdims=True)), 1e-12)
    t = jnp.sum(zn * zn, axis=1, keepdims=True)
    m = jax.lax.dot_general(zn, wn, (((1,), (1,)), ((), ())),
                            preferred_element_type=jnp.float32)
    d = (t + s_ref[...]) - 2.0 * m
    mind = jnp.min(d, axis=1)
    iota = jax.lax.broadcasted_iota(jnp.int32, d.shape, 1)
    idx = jnp.min(jnp.where(d == mind[:, None], iota, NUM_CODES), axis=1)
    idx_ref[0, 0, :] = idx

    oh = (iota == idx[:, None]).astype(jnp.float32)
    zq = jax.lax.dot_general(oh, wn, (((1,), (0,)), ((), ())),
                             preferred_element_type=jnp.float32,
                             precision=jax.lax.Precision.HIGHEST)
    zq_ref[...] = zn + (zq - zn)
    loss_ref[...] += jnp.sum(mind)[None, None]


@jax.jit
def _run(z, weight):
    wn = weight / jnp.maximum(jnp.linalg.norm(weight, axis=1, keepdims=True), 1e-12)
    s = jnp.sum(wn ** 2, axis=1)
    zq, idx3, losssum = pl.pallas_call(
        _body,
        grid=(G,),
        in_specs=[
            pl.BlockSpec((BN, LATENT_DIM), lambda i: (i, 0)),
            pl.BlockSpec((NUM_CODES, LATENT_DIM), lambda i: (0, 0)),
            pl.BlockSpec((1, NUM_CODES), lambda i: (0, 0)),
        ],
        out_specs=[
            pl.BlockSpec((BN, LATENT_DIM), lambda i: (i, 0)),
            pl.BlockSpec((1, 1, BN), lambda i: (i, 0, 0)),
            pl.BlockSpec((1, 1), lambda i: (0, 0)),
        ],
        out_shape=[
            jax.ShapeDtypeStruct((N_TOKENS, LATENT_DIM), jnp.float32),
            jax.ShapeDtypeStruct((G, 1, BN), jnp.int32),
            jax.ShapeDtypeStruct((1, 1), jnp.float32),
        ],
        compiler_params=pltpu.CompilerParams(
            dimension_semantics=("arbitrary",),
        ),
    )(z, wn, s[None, :])
    loss = losssum[0, 0] * ((1.0 + BETA) / (N_TOKENS * LATENT_DIM))
    return zq, idx3.reshape(N_TOKENS), loss


def kernel(z, weight):
    return _run(z, weight)


# speedup vs baseline: 1.5805x; 1.0849x over previous
import functools

import jax
import jax.numpy as jnp
from jax import lax
from jax.experimental import pallas as pl
from jax.experimental.pallas import tpu as pltpu
from jax.experimental.pallas import tpu_sc as plsc

NUM_CODES = 512
LATENT_DIM = 384
BETA = 0.25
N_TOKENS = 32768
BN = 1024
G = N_TOKENS // BN

_SC_INFO = plsc.get_sparse_core_info()
NC = _SC_INFO.num_cores
NS = _SC_INFO.num_subcores
NW = NC * NS
RPW = N_TOKENS // NW
CH = 128
NCHUNK = RPW // CH


def _tc_body(z_ref, wn_ref, s_ref, idx_ref, loss_ref):
    i = pl.program_id(0)

    @pl.when(i == 0)
    def _init():
        loss_ref[...] = jnp.zeros((1, 1), jnp.float32)

    wn = wn_ref[...]
    z = z_ref[...]
    zn = z / jnp.maximum(jnp.sqrt(jnp.sum(z * z, axis=1, keepdims=True)), 1e-12)
    t = jnp.sum(zn * zn, axis=1, keepdims=True)
    m = jax.lax.dot_general(zn, wn, (((1,), (1,)), ((), ())),
                            preferred_element_type=jnp.float32)
    d = (t + s_ref[...]) - 2.0 * m
    mind = jnp.min(d, axis=1)
    iota = jax.lax.broadcasted_iota(jnp.int32, d.shape, 1)
    idx = jnp.min(jnp.where(d == mind[:, None], iota, NUM_CODES), axis=1)
    idx_ref[0, 0, :] = idx
    loss_ref[...] += jnp.sum(mind)[None, None]


_sc_mesh = plsc.VectorSubcoreMesh(core_axis_name="c", subcore_axis_name="s")


@functools.partial(
    pl.kernel,
    mesh=_sc_mesh,
    out_type=jax.ShapeDtypeStruct((N_TOKENS, LATENT_DIM), jnp.float32),
    scratch_types=[
        pltpu.VMEM((CH,), jnp.int32),
        pltpu.VMEM((CH, LATENT_DIM), jnp.float32),
        pltpu.SemaphoreType.DMA,
    ],
)
def _sc_gather(wn_hbm, idx_hbm, out_hbm, idx_v, rows_v, sem):
    wid = lax.axis_index("s") * NC + lax.axis_index("c")
    base = wid * RPW
    for j in range(NCHUNK):
        off = base + j * CH
        pltpu.sync_copy(idx_hbm.at[pl.ds(off, CH)], idx_v)
        pltpu.async_copy(wn_hbm.at[idx_v], rows_v, sem).wait()
        pltpu.sync_copy(rows_v, out_hbm.at[pl.ds(off, CH)])


@jax.jit
def _run(z, weight):
    wn = weight / jnp.maximum(jnp.linalg.norm(weight, axis=1, keepdims=True), 1e-12)
    s = jnp.sum(wn ** 2, axis=1)
    idx3, losssum = pl.pallas_call(
        _tc_body,
        grid=(G,),
        in_specs=[
            pl.BlockSpec((BN, LATENT_DIM), lambda i: (i, 0)),
            pl.BlockSpec((NUM_CODES, LATENT_DIM), lambda i: (0, 0)),
            pl.BlockSpec((1, NUM_CODES), lambda i: (0, 0)),
        ],
        out_specs=[
            pl.BlockSpec((1, 1, BN), lambda i: (i, 0, 0)),
            pl.BlockSpec((1, 1), lambda i: (0, 0)),
        ],
        out_shape=[
            jax.ShapeDtypeStruct((G, 1, BN), jnp.int32),
            jax.ShapeDtypeStruct((1, 1), jnp.float32),
        ],
        compiler_params=pltpu.CompilerParams(
            dimension_semantics=("arbitrary",),
        ),
    )(z, wn, s[None, :])
    idx = idx3.reshape(N_TOKENS)
    zq = _sc_gather(wn, idx)
    loss = losssum[0, 0] * ((1.0 + BETA) / (N_TOKENS * LATENT_DIM))
    return zq, idx, loss


def kernel(z, weight):
    return _run(z, weight)


# speedup vs baseline: 1.6450x; 1.0408x over previous
import functools

import jax
import jax.numpy as jnp
from jax import lax
from jax.experimental import pallas as pl
from jax.experimental.pallas import tpu as pltpu
from jax.experimental.pallas import tpu_sc as plsc

NUM_CODES = 512
LATENT_DIM = 384
BETA = 0.25
N_TOKENS = 32768
BN = 1024
G = N_TOKENS // BN

_SC_INFO = plsc.get_sparse_core_info()
NC = _SC_INFO.num_cores
NS = _SC_INFO.num_subcores
NW = NC * NS
RPW = N_TOKENS // NW
CH = 128
NCHUNK = RPW // CH


def _tc_body(z_ref, wn_ref, s_ref, idx_ref, loss_ref):
    i = pl.program_id(0)

    @pl.when(i == 0)
    def _init():
        loss_ref[...] = jnp.zeros((1, 1), jnp.float32)

    wn = wn_ref[...]
    z = z_ref[...]
    zn = z / jnp.maximum(jnp.sqrt(jnp.sum(z * z, axis=1, keepdims=True)), 1e-12)
    t = jnp.sum(zn * zn, axis=1, keepdims=True)
    m = jax.lax.dot_general(zn, wn, (((1,), (1,)), ((), ())),
                            preferred_element_type=jnp.float32)
    d = (t + s_ref[...]) - 2.0 * m
    mind = jnp.min(d, axis=1)
    iota = jax.lax.broadcasted_iota(jnp.int32, d.shape, 1)
    idx = jnp.min(jnp.where(d == mind[:, None], iota, NUM_CODES), axis=1)
    idx_ref[0, 0, :] = idx
    loss_ref[...] += jnp.sum(mind)[None, None]


_sc_mesh = plsc.VectorSubcoreMesh(core_axis_name="c", subcore_axis_name="s")


@functools.partial(
    pl.kernel,
    mesh=_sc_mesh,
    out_type=jax.ShapeDtypeStruct((N_TOKENS, LATENT_DIM), jnp.float32),
    scratch_types=[
        pltpu.VMEM((CH,), jnp.int32),
        pltpu.VMEM((CH,), jnp.int32),
        pltpu.VMEM((CH, LATENT_DIM), jnp.float32),
        pltpu.VMEM((CH, LATENT_DIM), jnp.float32),
        pltpu.SemaphoreType.DMA,
        pltpu.SemaphoreType.DMA,
    ],
)
def _sc_gather(wn_hbm, idx_hbm, out_hbm, idx_v0, idx_v1, rows_v0, rows_v1,
               sem0, sem1):
    wid = lax.axis_index("s") * NC + lax.axis_index("c")
    base = wid * RPW
    idx_bufs = (idx_v0, idx_v1)
    row_bufs = (rows_v0, rows_v1)
    sems = (sem0, sem1)
    handles = [None, None]
    pltpu.sync_copy(idx_hbm.at[pl.ds(base, CH)], idx_bufs[0])
    handles[0] = pltpu.async_copy(wn_hbm.at[idx_bufs[0]], row_bufs[0], sems[0])
    for j in range(NCHUNK):
        b, nb = j % 2, (j + 1) % 2
        if j + 1 < NCHUNK:
            off = base + (j + 1) * CH
            pltpu.sync_copy(idx_hbm.at[pl.ds(off, CH)], idx_bufs[nb])
            handles[nb] = pltpu.async_copy(
                wn_hbm.at[idx_bufs[nb]], row_bufs[nb], sems[nb])
        handles[b].wait()
        pltpu.sync_copy(row_bufs[b], out_hbm.at[pl.ds(base + j * CH, CH)])


@jax.jit
def _run(z, weight):
    wn = weight / jnp.maximum(jnp.linalg.norm(weight, axis=1, keepdims=True), 1e-12)
    s = jnp.sum(wn ** 2, axis=1)
    idx3, losssum = pl.pallas_call(
        _tc_body,
        grid=(G,),
        in_specs=[
            pl.BlockSpec((BN, LATENT_DIM), lambda i: (i, 0)),
            pl.BlockSpec((NUM_CODES, LATENT_DIM), lambda i: (0, 0)),
            pl.BlockSpec((1, NUM_CODES), lambda i: (0, 0)),
        ],
        out_specs=[
            pl.BlockSpec((1, 1, BN), lambda i: (i, 0, 0)),
            pl.BlockSpec((1, 1), lambda i: (0, 0)),
        ],
        out_shape=[
            jax.ShapeDtypeStruct((G, 1, BN), jnp.int32),
            jax.ShapeDtypeStruct((1, 1), jnp.float32),
        ],
        compiler_params=pltpu.CompilerParams(
            dimension_semantics=("arbitrary",),
        ),
    )(z, wn, s[None, :])
    idx = idx3.reshape(N_TOKENS)
    zq = _sc_gather(wn, idx)
    loss = losssum[0, 0] * ((1.0 + BETA) / (N_TOKENS * LATENT_DIM))
    return zq, idx, loss


def kernel(z, weight):
    return _run(z, weight)
